# Initial kernel scaffold; baseline (speedup 1.0000x reference)
import jax
import jax.numpy as jnp
from jax.experimental import pallas as pl


def kernel(edge_index, edge_vals, user_embeds, item_embeds):
    raise NotImplementedError("write your pallas kernel here")


# speedup vs baseline: 2.6610x; 2.6610x over previous
import functools

import jax
import jax.numpy as jnp
from jax import lax
from jax.experimental import pallas as pl
from jax.experimental.pallas import tpu as pltpu
from jax.experimental.pallas import tpu_sc as plsc

USER_NUM = 2000
ITEM_NUM = 8000
LATDIM = 256
GCN_LAYER = 3
N_EDGES = 160000
N_NODES = USER_NUM + ITEM_NUM

NSUB = 16
DHALF = LATDIM // 2
NJ = DHALF // 16
K = 80
EPT = N_EDGES // NSUB
NCHUNK = EPT // K
NPAD = 10240
RPT = NPAD // NSUB
RBLK = 64
NRB = RPT // RBLK


def _gcn_body(row_hbm, col_hbm, val_hbm, tin_hbm,
              f_hbm, tscr_hbm,
              acc, colbuf, rowbuf, valbuf, rowsbuf, abuf, fbuf, sem):
    c = lax.axis_index("c")
    s = lax.axis_index("s")
    coff = c * NPAD
    ebase = s * EPT
    rbase = c * NPAD + s * RPT

    zv = jnp.zeros((16,), jnp.float32)

    for l in range(GCN_LAYER):
        src = tin_hbm if l == 0 else tscr_hbm

        def zinit(i, carry):
            for j in range(NJ):
                abuf[i, pl.ds(j * 16, 16)] = zv
            return carry

        lax.fori_loop(0, RBLK, zinit, 0)

        def zero_blk(b, carry):
            pltpu.sync_copy(abuf, acc.at[pl.ds(s * RPT + b * RBLK, RBLK)])
            return carry

        lax.fori_loop(0, NRB, zero_blk, 0)
        plsc.subcore_barrier()

        def chunk(k, carry):
            base = pl.multiple_of(ebase + k * K, 8)
            pltpu.sync_copy(col_hbm.at[pl.ds(base, K)], colbuf)
            pltpu.sync_copy(row_hbm.at[pl.ds(base, K)], rowbuf)
            pltpu.sync_copy(val_hbm.at[pl.ds(base, K)], valbuf)
            for g in range(K // 16):
                colbuf[pl.ds(g * 16, 16)] = colbuf[pl.ds(g * 16, 16)] + coff
            pltpu.async_copy(src.at[colbuf], rowsbuf, sem).wait()

            def scale(g, ecarry):
                vv = valbuf[pl.ds(g * 16, 16)]
                for e16 in range(16):
                    e = g * 16 + e16
                    sv = vv[e16]
                    for j in range(NJ):
                        rowsbuf[e, pl.ds(j * 16, 16)] = (
                            rowsbuf[e, pl.ds(j * 16, 16)] * sv)
                return ecarry

            lax.fori_loop(0, K // 16, scale, 0)
            pltpu.sync_copy(rowsbuf, acc.at[rowbuf], add=True)
            return carry

        lax.fori_loop(0, NCHUNK, chunk, 0)
        plsc.subcore_barrier()

        fsrc = tin_hbm if l == 0 else f_hbm

        def readback(b, carry):
            r0 = s * RPT + b * RBLK
            g0 = rbase + b * RBLK
            pltpu.sync_copy(acc.at[pl.ds(r0, RBLK)], abuf)
            pltpu.sync_copy(fsrc.at[pl.ds(g0, RBLK)], fbuf)

            def addrow(i, icarry):
                for j in range(NJ):
                    fbuf[i, pl.ds(j * 16, 16)] = (
                        fbuf[i, pl.ds(j * 16, 16)]
                        + abuf[i, pl.ds(j * 16, 16)])
                return icarry

            lax.fori_loop(0, RBLK, addrow, 0)
            pltpu.sync_copy(fbuf, f_hbm.at[pl.ds(g0, RBLK)])
            if l < GCN_LAYER - 1:
                pltpu.sync_copy(abuf, tscr_hbm.at[pl.ds(g0, RBLK)])
            return carry

        lax.fori_loop(0, NRB, readback, 0)
        if l < GCN_LAYER - 1:
            plsc.subcore_barrier()


@jax.jit
def _run(row, col, vals, tin):
    mesh = plsc.VectorSubcoreMesh(core_axis_name="c", subcore_axis_name="s")
    fn = functools.partial(
        pl.kernel,
        mesh=mesh,
        out_type=(
            jax.ShapeDtypeStruct((2 * NPAD, DHALF), jnp.float32),
            jax.ShapeDtypeStruct((2 * NPAD, DHALF), jnp.float32),
        ),
        scratch_types=[
            pltpu.VMEM_SHARED((NPAD, DHALF), jnp.float32),
            pltpu.VMEM((K,), jnp.int32),
            pltpu.VMEM((K,), jnp.int32),
            pltpu.VMEM((K,), jnp.float32),
            pltpu.VMEM((K, DHALF), jnp.float32),
            pltpu.VMEM((RBLK, DHALF), jnp.float32),
            pltpu.VMEM((RBLK, DHALF), jnp.float32),
            pltpu.SemaphoreType.DMA,
        ],
    )(_gcn_body)
    f, _ = fn(row, col, vals, tin)
    return f


def kernel(edge_index, edge_vals, user_embeds, item_embeds):
    embeds = jnp.concatenate([user_embeds, item_embeds], axis=0)
    tin = (jnp.zeros((2 * NPAD, DHALF), jnp.float32)
           .at[:N_NODES].set(embeds[:, :DHALF])
           .at[NPAD:NPAD + N_NODES].set(embeds[:, DHALF:]))
    f = _run(edge_index[0], edge_index[1], edge_vals, tin)
    final = jnp.concatenate(
        [f[:N_NODES], f[NPAD:NPAD + N_NODES]], axis=1)
    return final[:USER_NUM], final[USER_NUM:]


# speedup vs baseline: 2.9548x; 1.1104x over previous
import functools

import jax
import jax.numpy as jnp
from jax import lax
from jax.experimental import pallas as pl
from jax.experimental.pallas import tpu as pltpu
from jax.experimental.pallas import tpu_sc as plsc

USER_NUM = 2000
ITEM_NUM = 8000
LATDIM = 256
GCN_LAYER = 3
N_EDGES = 160000
N_NODES = USER_NUM + ITEM_NUM

NSUB = 16
DHALF = LATDIM // 2
NJ = DHALF // 16
K = 128
EPT = 10240
EPAD = NSUB * EPT
NCHUNK = EPT // K
NPAD = 10240
RPT = NPAD // NSUB
RBLK = 32
NRB = RPT // RBLK


def _gcn_body(cpk_hbm, vpk_hbm, rpk_hbm, tin_hbm,
              f_hbm, tscr_hbm,
              acc, cbuf, vbuf, rbuf, rs0, rs1, abuf, fbuf,
              semi, semg, semsc):
    c = lax.axis_index("c")
    s = lax.axis_index("s")
    rbase = c * NPAD + s * RPT

    zv = jnp.zeros((16,), jnp.float32)

    def start_idx(k, slot):
        pltpu.async_copy(cpk_hbm.at[c, s, k], cbuf.at[slot], semi)
        pltpu.async_copy(vpk_hbm.at[s, k], vbuf.at[slot], semi)
        pltpu.async_copy(rpk_hbm.at[s, k], rbuf.at[slot], semi)

    def wait_idx(slot):
        pltpu.make_async_copy(cpk_hbm.at[c, s, 0], cbuf.at[slot], semi).wait()
        pltpu.make_async_copy(vpk_hbm.at[s, 0], vbuf.at[slot], semi).wait()
        pltpu.make_async_copy(rpk_hbm.at[s, 0], rbuf.at[slot], semi).wait()

    def start_gather(slot, src):
        rs = rs0 if slot == 0 else rs1
        pltpu.async_copy(src.at[cbuf.at[slot]], rs, semg)

    def wait_gather(slot, src):
        rs = rs0 if slot == 0 else rs1
        pltpu.make_async_copy(src.at[cbuf.at[slot]], rs, semg).wait()

    def start_scatter(slot):
        rs = rs0 if slot == 0 else rs1
        pltpu.async_copy(rs, acc.at[rbuf.at[slot]], semsc, add=True)

    def wait_scatter(slot):
        rs = rs0 if slot == 0 else rs1
        pltpu.make_async_copy(rs, acc.at[rbuf.at[slot]], semsc).wait()

    def scale_chunk(slot):
        rs = rs0 if slot == 0 else rs1

        def scale(g, ecarry):
            vv = vbuf[slot, pl.ds(g * 16, 16)]
            for e16 in range(16):
                e = g * 16 + e16
                sv = vv[e16]
                for j in range(NJ):
                    rs[e, pl.ds(j * 16, 16)] = rs[e, pl.ds(j * 16, 16)] * sv
            return ecarry

        lax.fori_loop(0, K // 16, scale, 0)

    for l in range(GCN_LAYER):
        src = tin_hbm if l == 0 else tscr_hbm

        def zinit(i, carry):
            for j in range(NJ):
                abuf[i, pl.ds(j * 16, 16)] = zv
            return carry

        lax.fori_loop(0, RBLK, zinit, 0)

        def zero_blk(b, carry):
            pltpu.sync_copy(abuf, acc.at[pl.ds(s * RPT + b * RBLK, RBLK)])
            return carry

        lax.fori_loop(0, NRB, zero_blk, 0)
        plsc.subcore_barrier()

        start_idx(0, 0)
        wait_idx(0)
        start_gather(0, src)

        def pipe(i, carry):
            for b in range(2):
                k = 2 * i + b
                wait_gather(b, src)

                @pl.when(k >= 1)
                def _():
                    wait_scatter(1 - b)

                @pl.when(k + 1 < NCHUNK)
                def _():
                    start_idx(k + 1, 1 - b)
                    wait_idx(1 - b)
                    start_gather(1 - b, src)

                scale_chunk(b)
                start_scatter(b)
            return carry

        lax.fori_loop(0, NCHUNK // 2, pipe, 0)
        wait_scatter(1)
        plsc.subcore_barrier()

        fsrc = tin_hbm if l == 0 else f_hbm

        def readback(b, carry):
            r0 = s * RPT + b * RBLK
            g0 = rbase + b * RBLK
            pltpu.sync_copy(acc.at[pl.ds(r0, RBLK)], abuf)
            pltpu.sync_copy(fsrc.at[pl.ds(g0, RBLK)], fbuf)

            def addrow(i, icarry):
                for j in range(NJ):
                    fbuf[i, pl.ds(j * 16, 16)] = (
                        fbuf[i, pl.ds(j * 16, 16)]
                        + abuf[i, pl.ds(j * 16, 16)])
                return icarry

            lax.fori_loop(0, RBLK, addrow, 0)
            pltpu.sync_copy(fbuf, f_hbm.at[pl.ds(g0, RBLK)])
            if l < GCN_LAYER - 1:
                pltpu.sync_copy(abuf, tscr_hbm.at[pl.ds(g0, RBLK)])
            return carry

        lax.fori_loop(0, NRB, readback, 0)
        if l < GCN_LAYER - 1:
            plsc.subcore_barrier()


@jax.jit
def _run(cpk, vpk, rpk, tin):
    mesh = plsc.VectorSubcoreMesh(core_axis_name="c", subcore_axis_name="s")
    fn = functools.partial(
        pl.kernel,
        mesh=mesh,
        out_type=(
            jax.ShapeDtypeStruct((2 * NPAD, DHALF), jnp.float32),
            jax.ShapeDtypeStruct((2 * NPAD, DHALF), jnp.float32),
        ),
        scratch_types=[
            pltpu.VMEM_SHARED((NPAD, DHALF), jnp.float32),
            pltpu.VMEM((2, K), jnp.int32),
            pltpu.VMEM((2, K), jnp.float32),
            pltpu.VMEM((2, K), jnp.int32),
            pltpu.VMEM((K, DHALF), jnp.float32),
            pltpu.VMEM((K, DHALF), jnp.float32),
            pltpu.VMEM((RBLK, DHALF), jnp.float32),
            pltpu.VMEM((RBLK, DHALF), jnp.float32),
            pltpu.SemaphoreType.DMA,
            pltpu.SemaphoreType.DMA,
            pltpu.SemaphoreType.DMA,
        ],
    )(_gcn_body)
    f, _ = fn(cpk, vpk, rpk, tin)
    return f


def kernel(edge_index, edge_vals, user_embeds, item_embeds):
    embeds = jnp.concatenate([user_embeds, item_embeds], axis=0)
    tin = (jnp.zeros((2 * NPAD, DHALF), jnp.float32)
           .at[:N_NODES].set(embeds[:, :DHALF])
           .at[NPAD:NPAD + N_NODES].set(embeds[:, DHALF:]))
    pad = EPAD - N_EDGES
    row = jnp.pad(edge_index[0], (0, pad)).reshape(NSUB, NCHUNK, K)
    col = jnp.pad(edge_index[1], (0, pad)).reshape(NSUB, NCHUNK, K)
    vpk = jnp.pad(edge_vals, (0, pad)).reshape(NSUB, NCHUNK, K)
    cpk = jnp.stack([col, col + NPAD], axis=0)
    f = _run(cpk, vpk, row, tin)
    final = jnp.concatenate(
        [f[:N_NODES], f[NPAD:NPAD + N_NODES]], axis=1)
    return final[:USER_NUM], final[USER_NUM:]
